# baseline (speedup 1.0000x reference)
import functools

import jax
import jax.numpy as jnp
from jax import lax
from jax.experimental import pallas as pl
from jax.experimental.pallas import tpu as pltpu
from jax.experimental.pallas import tpu_sc as plsc

N_NODES = 10000
D = 128
NC = 2
NS = 16
NW = NC * NS
K = 128
RB = 2
MR = 4
ACC_ROWS = 10240
STRIPE = ACC_ROWS // NS


def _make_scatter(nchunk):
  mesh = plsc.VectorSubcoreMesh(core_axis_name="c", subcore_axis_name="s",
                                num_cores=NC, num_subcores=NS)

  @functools.partial(
      pl.kernel,
      out_type=jax.ShapeDtypeStruct((NC, ACC_ROWS, D), jnp.float32),
      mesh=mesh,
      scratch_types=[
          pltpu.VMEM((MR * 3, K), jnp.int32),
          pltpu.VMEM((RB, K, D), jnp.float32),
          pltpu.VMEM_SHARED((ACC_ROWS, D), jnp.float32),
          [pltpu.SemaphoreType.DMA] * MR,
          [pltpu.SemaphoreType.DMA] * RB,
          [pltpu.SemaphoreType.DMA] * RB,
      ],
  )
  def scatter(pre_hbm, meta_hbm, z_hbm, out_hbm,
              meta, rows, acc, msems, gsems, ssems):
    cid = lax.axis_index("c")
    sid = lax.axis_index("s")
    wid = sid * NC + cid
    pltpu.sync_copy(z_hbm, acc.at[pl.ds(sid * STRIPE, STRIPE)])
    plsc.subcore_barrier()

    def scale(buf, m):
      def group(g, carry2):
        ew16 = lax.bitcast_convert_type(meta[3 * m + 2, pl.ds(g * 16, 16)],
                                        jnp.float32)
        for j in range(16):
          w = ew16[j]
          e = g * 16 + j
          for f in range(D // 16):
            sl = pl.ds(f * 16, 16)
            buf[e, sl] = buf[e, sl] * w
        return carry2

      lax.fori_loop(0, K // 16, group, 0)

    for j in range(3):
      pltpu.async_copy(meta_hbm.at[wid, j], meta.at[pl.ds(3 * j, 3)], msems[j])
    pltpu.make_async_copy(meta_hbm.at[wid, 0], meta.at[pl.ds(0, 3)], msems[0]).wait()
    pltpu.async_copy(pre_hbm.at[meta.at[0]], rows.at[0], gsems[0])

    def outer(i, carry):
      base = MR * i
      for j in range(MR):
        c = base + j
        jn, jc = (j + 1) % RB, j % RB
        mn, mc, mp = (j + 1) % MR, j % MR, (j + 3) % MR

        @pl.when((c >= 1) & (c + 1 < nchunk))
        def _wait_prev_scatter():
          pltpu.make_async_copy(rows.at[jn], acc.at[meta.at[3 * mp + 1]],
                                ssems[jn]).wait()

        @pl.when(c + 1 < nchunk)
        def _prefetch_rows():
          pltpu.make_async_copy(meta_hbm.at[wid, c + 1],
                                meta.at[pl.ds(3 * mn, 3)], msems[mn]).wait()
          pltpu.async_copy(pre_hbm.at[meta.at[3 * mn]], rows.at[jn], gsems[jn])

        @pl.when(c + 3 < nchunk)
        def _prefetch_meta():
          pltpu.async_copy(meta_hbm.at[wid, c + 3],
                           meta.at[pl.ds(3 * mp, 3)], msems[mp])

        pltpu.make_async_copy(pre_hbm.at[meta.at[3 * mc]], rows.at[jc],
                              gsems[jc]).wait()
        scale(rows.at[jc], mc)
        pltpu.async_copy(rows.at[jc], acc.at[meta.at[3 * mc + 1]], ssems[jc],
                         add=True)
      return carry

    lax.fori_loop(0, nchunk // MR, outer, 0)
    pltpu.make_async_copy(rows.at[0], acc.at[meta.at[3 * (MR - 2) + 1]],
                          ssems[0]).wait()
    pltpu.make_async_copy(rows.at[1], acc.at[meta.at[3 * (MR - 1) + 1]],
                          ssems[1]).wait()
    plsc.subcore_barrier()
    pltpu.sync_copy(acc.at[pl.ds(sid * STRIPE, STRIPE)],
                    out_hbm.at[cid, pl.ds(sid * STRIPE, STRIPE)])

  return scatter


def _mm_plain_body(x_ref, w_ref, o_ref):
  o_ref[...] = jnp.dot(x_ref[...], w_ref[...],
                       preferred_element_type=jnp.float32)


def _mm_fused_body(a_ref, b_ref, w_ref, o_ref):
  h = jnp.maximum(a_ref[...] + b_ref[...], 0.0)
  o_ref[...] = jnp.dot(h, w_ref[...], preferred_element_type=jnp.float32)


def _mm_fused_bias_body(a_ref, b_ref, w_ref, bias_ref, o_ref):
  h = jnp.maximum(a_ref[...] + b_ref[...], 0.0)
  o_ref[...] = (jnp.dot(h, w_ref[...], preferred_element_type=jnp.float32)
                + bias_ref[...])


_BM = 2000


def _matmul(x, w):
  m, k = x.shape
  n = w.shape[1]
  return pl.pallas_call(
      _mm_plain_body,
      grid=(m // _BM,),
      in_specs=[pl.BlockSpec((_BM, k), lambda i: (i, 0)),
                pl.BlockSpec((k, n), lambda i: (0, 0))],
      out_specs=pl.BlockSpec((_BM, n), lambda i: (i, 0)),
      out_shape=jax.ShapeDtypeStruct((m, n), jnp.float32),
  )(x, w)


def _fused_matmul(a, b, w):
  m, k = a.shape
  n = w.shape[1]
  return pl.pallas_call(
      _mm_fused_body,
      grid=(m // _BM,),
      in_specs=[pl.BlockSpec((_BM, k), lambda i: (i, 0)),
                pl.BlockSpec((_BM, k), lambda i: (i, 0)),
                pl.BlockSpec((k, n), lambda i: (0, 0))],
      out_specs=pl.BlockSpec((_BM, n), lambda i: (i, 0)),
      out_shape=jax.ShapeDtypeStruct((m, n), jnp.float32),
  )(a, b, w)


def _fused_matmul_bias(a, b, w, bias):
  m, k = a.shape
  n = w.shape[1]
  return pl.pallas_call(
      _mm_fused_bias_body,
      grid=(m // _BM,),
      in_specs=[pl.BlockSpec((_BM, k), lambda i: (i, 0)),
                pl.BlockSpec((_BM, k), lambda i: (i, 0)),
                pl.BlockSpec((k, n), lambda i: (0, 0)),
                pl.BlockSpec((1, n), lambda i: (0, 0))],
      out_specs=pl.BlockSpec((_BM, n), lambda i: (i, 0)),
      out_shape=jax.ShapeDtypeStruct((m, n), jnp.float32),
  )(a, b, w, bias)


def kernel(x, edge_index, edge_weight, W0, W1, Wp, bp):
  n_edges = edge_index.shape[1]
  grain = K * MR
  ept = ((n_edges + NW * grain - 1) // (NW * grain)) * grain
  nchunk = ept // K
  pad = NW * ept - n_edges

  src = jnp.pad(edge_index[0].astype(jnp.int32), (0, pad))
  dst = jnp.pad(edge_index[1].astype(jnp.int32), (0, pad))
  ew = lax.bitcast_convert_type(
      jnp.pad(edge_weight.astype(jnp.float32), (0, pad)), jnp.int32)
  meta = jnp.stack([src.reshape(NW, nchunk, K),
                    dst.reshape(NW, nchunk, K),
                    ew.reshape(NW, nchunk, K)], axis=2)
  zeros = jnp.zeros((STRIPE, D), jnp.float32)

  scatter = _make_scatter(nchunk)

  n = x.shape[0]
  pre0 = _matmul(x, W0)
  p = scatter(pre0, meta, zeros)
  pre1 = _fused_matmul(p[0, :n], p[1, :n], W1)
  q = scatter(pre1, meta, zeros)

  out_dim = Wp.shape[1]
  wp = jnp.pad(Wp, ((0, 0), (0, D - out_dim)))
  bpad = jnp.pad(bp, (0, D - out_dim)).reshape(1, D)
  out = _fused_matmul_bias(q[0, :n], q[1, :n], wp, bpad)
  return out[:, :out_dim]
